# Initial kernel scaffold; baseline (speedup 1.0000x reference)
import jax
import jax.numpy as jnp
from jax.experimental import pallas as pl


def kernel(x, edge_index, W, b, gamma, beta):
    raise NotImplementedError("write your pallas kernel here")


# speedup vs baseline: 5.0643x; 5.0643x over previous
import functools
import math

import jax
import jax.numpy as jnp
from jax import lax
from jax.experimental import pallas as pl
from jax.experimental.pallas import tpu as pltpu
from jax.experimental.pallas import tpu_sc as plsc

N = 10000
K = 32
C = 128
COUT = 128

NW = 32
NPW = 320
NPAD = NW * NPW
CHUNK = 4
NCHUNK = NPW // CHUNK
LANES = 16
CB = C // LANES

_sc_mesh = plsc.VectorSubcoreMesh(core_axis_name="c", subcore_axis_name="s")


@functools.partial(
    pl.kernel,
    mesh=_sc_mesh,
    out_type=jax.ShapeDtypeStruct((NPAD, C), jnp.float32),
    scratch_types=[
        pltpu.VMEM((NW * NCHUNK // NW, 128), jnp.int32),
        pltpu.VMEM((2, CHUNK * K, C), jnp.float32),
        pltpu.VMEM((NPW, C), jnp.float32),
        pltpu.SemaphoreType.DMA,
        pltpu.SemaphoreType.DMA,
    ],
)
def _sc_gather_max(x_hbm, e_hbm, out_hbm, e_v, rows_v, out_v, sem0, sem1):
    wid = lax.axis_index("s") * 2 + lax.axis_index("c")
    pltpu.sync_copy(e_hbm.at[pl.ds(wid * NCHUNK, NCHUNK)], e_v)
    sems = (sem0, sem1)

    def gather(step, buf):
        return pltpu.make_async_copy(
            x_hbm.at[e_v.at[step]], rows_v.at[buf], sems[buf])

    gather(0, 0).start()
    gather(1, 1).start()

    def compute(step, buf):
        for n in range(CHUNK):
            accs = tuple(
                rows_v[buf, n * K, pl.ds(cb * LANES, LANES)] for cb in range(CB))

            def body(k, accs, n=n):
                return tuple(
                    jnp.maximum(a, rows_v[buf, n * K + k, pl.ds(cb * LANES, LANES)])
                    for cb, a in enumerate(accs))

            accs = lax.fori_loop(1, K, body, accs)
            for cb in range(CB):
                out_v[step * CHUNK + n, pl.ds(cb * LANES, LANES)] = accs[cb]

    def loop_body(g, carry):
        for buf in range(2):
            step = g * 2 + buf
            gather(step, buf).wait()
            compute(step, buf)
            nxt = step + 2

            @pl.when(nxt < NCHUNK)
            def _():
                gather(nxt, buf).start()
        return carry

    lax.fori_loop(0, NCHUNK // 2, loop_body, 0)
    pltpu.sync_copy(out_v, out_hbm.at[pl.ds(wid * NPW, NPW)])


def _erf(z):
    a1, a2, a3, a4, a5 = (0.254829592, -0.284496736, 1.421413741,
                          -1.453152027, 1.061405429)
    p = 0.3275911
    s = jnp.sign(z)
    az = jnp.abs(z)
    t = 1.0 / (1.0 + p * az)
    poly = ((((a5 * t + a4) * t + a3) * t + a2) * t + a1) * t
    return s * (1.0 - poly * jnp.exp(-az * az))


def _tc_mlp_body(x_ref, g_ref, wd_ref, w2_ref, b_ref, gm_ref, bt_ref, o_ref):
    h = (jnp.dot(x_ref[...], wd_ref[...], preferred_element_type=jnp.float32)
         + jnp.dot(g_ref[...], w2_ref[...], preferred_element_type=jnp.float32)
         + b_ref[...])
    mean = jnp.mean(h, axis=0, keepdims=True)
    var = jnp.mean((h - mean) ** 2, axis=0, keepdims=True)
    hn = (h - mean) * lax.rsqrt(var + 1e-5) * gm_ref[...] + bt_ref[...]
    o_ref[...] = 0.5 * hn * (1.0 + _erf(hn * (1.0 / math.sqrt(2.0))))


def kernel(x, edge_index, W, b, gamma, beta):
    xf = x[0]
    e = edge_index[0]
    e_pad = jnp.concatenate(
        [e, jnp.zeros((NPAD - N, K), jnp.int32)], axis=0)
    e2 = e_pad.reshape(NW * NCHUNK, 128)

    maxg = _sc_gather_max(xf, e2)[:N]

    wd = W[:C] - W[C:]
    w2 = W[C:]
    out = pl.pallas_call(
        _tc_mlp_body,
        out_shape=jax.ShapeDtypeStruct((N, COUT), jnp.float32),
    )(xf, maxg, wd, w2, b.reshape(1, COUT), gamma.reshape(1, COUT),
      beta.reshape(1, COUT))
    return out.reshape(1, N, COUT)


# speedup vs baseline: 22.9844x; 4.5385x over previous
import functools
import math

import jax
import jax.numpy as jnp
from jax import lax
from jax.experimental import pallas as pl
from jax.experimental.pallas import tpu as pltpu
from jax.experimental.pallas import tpu_sc as plsc

N = 10000
K = 32
C = 128
COUT = 128

NW = 32
NPW = 320
NPAD = NW * NPW
CHUNK = 2
NCHUNK = NPW // CHUNK
NBUF = 2
OROWS = 8
OSTEPS = OROWS // CHUNK
LANES = 16
CB = C // LANES

_sc_mesh = plsc.VectorSubcoreMesh(core_axis_name="c", subcore_axis_name="s")


@functools.partial(
    pl.kernel,
    mesh=_sc_mesh,
    out_type=jax.ShapeDtypeStruct((NPAD, C), jnp.float32),
    scratch_types=[
        pltpu.VMEM((NCHUNK, CHUNK * K), jnp.int32),
        pltpu.VMEM((NBUF, CHUNK * K, C), jnp.float32),
        pltpu.VMEM((2, OROWS, C), jnp.float32),
        pltpu.VMEM_SHARED((NPAD, C), jnp.float32),
        pltpu.SemaphoreType.DMA,
        pltpu.SemaphoreType.DMA,
        pltpu.SemaphoreType.DMA,
        pltpu.SemaphoreType.DMA,
    ],
)
def _sc_gather_max(x_hbm, e_hbm, out_hbm, e_v, rows_v, o_v, xs,
                   g0, g1, os0, os1):
    wid = lax.axis_index("s") * 2 + lax.axis_index("c")
    sid = lax.axis_index("s")
    rpw = NPAD // 16
    pltpu.sync_copy(x_hbm.at[pl.ds(sid * rpw, rpw)], xs.at[pl.ds(sid * rpw, rpw)])
    pltpu.sync_copy(e_hbm.at[pl.ds(wid * NCHUNK, NCHUNK)], e_v)
    plsc.subcore_barrier()
    gsems = (g0, g1)
    osems = (os0, os1)

    def gather(step, buf):
        return pltpu.make_async_copy(
            xs.at[e_v.at[step]], rows_v.at[buf], gsems[buf])

    def flush(octet, ob):
        return pltpu.make_async_copy(
            o_v.at[ob],
            out_hbm.at[pl.ds(wid * NPW + octet * OROWS, OROWS)],
            osems[ob])

    for b in range(NBUF):
        gather(b, b).start()

    def compute(step, buf, orow):
        for n in range(CHUNK):
            accs = tuple(
                rows_v[buf, n * K, pl.ds(cb * LANES, LANES)] for cb in range(CB))

            def body(k, accs, n=n, buf=buf):
                return tuple(
                    jnp.maximum(a, rows_v[buf, n * K + k, pl.ds(cb * LANES, LANES)])
                    for cb, a in enumerate(accs))

            accs = lax.fori_loop(1, K, body, accs)
            for cb in range(CB):
                o_v[orow // OROWS, orow % OROWS + n, pl.ds(cb * LANES, LANES)] = accs[cb]

    def loop_body(gg, carry):
        for h in range(2):
            octet = gg * 2 + h

            @pl.when(gg >= 1)
            def _(h=h):
                flush(octet - 2, h).wait()

            for j in range(OSTEPS):
                step = octet * OSTEPS + j
                buf = (4 * h + j) % NBUF
                gather(step, buf).wait()
                compute(step, buf, h * OROWS + j * CHUNK)
                nxt = step + NBUF

                @pl.when(nxt < NCHUNK)
                def _(nxt=nxt, buf=buf):
                    gather(nxt, buf).start()
            flush(octet, h).start()
        return carry

    lax.fori_loop(0, NCHUNK // (2 * OSTEPS), loop_body, 0)
    flush(NCHUNK // OSTEPS - 2, 0).wait()
    flush(NCHUNK // OSTEPS - 1, 1).wait()


def _erf(z):
    a1, a2, a3, a4, a5 = (0.254829592, -0.284496736, 1.421413741,
                          -1.453152027, 1.061405429)
    p = 0.3275911
    s = jnp.sign(z)
    az = jnp.abs(z)
    t = 1.0 / (1.0 + p * az)
    poly = ((((a5 * t + a4) * t + a3) * t + a2) * t + a1) * t
    return s * (1.0 - poly * jnp.exp(-az * az))


def _tc_mlp_body(x_ref, g_ref, wd_ref, w2_ref, b_ref, gm_ref, bt_ref, o_ref):
    h = (jnp.dot(x_ref[...], wd_ref[...], preferred_element_type=jnp.float32)
         + jnp.dot(g_ref[...], w2_ref[...], preferred_element_type=jnp.float32)
         + b_ref[...])
    mean = jnp.mean(h, axis=0, keepdims=True)
    var = jnp.mean((h - mean) ** 2, axis=0, keepdims=True)
    hn = (h - mean) * lax.rsqrt(var + 1e-5) * gm_ref[...] + bt_ref[...]
    o_ref[...] = 0.5 * hn * (1.0 + _erf(hn * (1.0 / math.sqrt(2.0))))


def kernel(x, edge_index, W, b, gamma, beta):
    xf = x[0]
    e = edge_index[0]
    e_pad = jnp.concatenate(
        [e, jnp.zeros((NPAD - N, K), jnp.int32)], axis=0)
    e2 = e_pad.reshape(NW * NCHUNK, CHUNK * K)
    x_pad = jnp.concatenate(
        [xf, jnp.zeros((NPAD - N, C), jnp.float32)], axis=0)

    maxg = _sc_gather_max(x_pad, e2)[:N]

    wd = W[:C] - W[C:]
    w2 = W[C:]
    out = pl.pallas_call(
        _tc_mlp_body,
        out_shape=jax.ShapeDtypeStruct((N, COUT), jnp.float32),
    )(xf, maxg, wd, w2, b.reshape(1, COUT), gamma.reshape(1, COUT),
      beta.reshape(1, COUT))
    return out.reshape(1, N, COUT)


# speedup vs baseline: 23.5179x; 1.0232x over previous
import functools
import math

import jax
import jax.numpy as jnp
from jax import lax
from jax.experimental import pallas as pl
from jax.experimental.pallas import tpu as pltpu
from jax.experimental.pallas import tpu_sc as plsc

N = 10000
K = 32
C = 128
COUT = 128

NW = 32
NPW = 320
NPAD = NW * NPW
CHUNK = 2
NCHUNK = NPW // CHUNK
NBUF = 2
OROWS = 8
OSTEPS = OROWS // CHUNK
LANES = 16
CB = C // LANES

_sc_mesh = plsc.VectorSubcoreMesh(core_axis_name="c", subcore_axis_name="s")


@functools.partial(
    pl.kernel,
    mesh=_sc_mesh,
    out_type=jax.ShapeDtypeStruct((N, C), jnp.float32),
    scratch_types=[
        pltpu.VMEM((NCHUNK, CHUNK * K), jnp.int32),
        pltpu.VMEM((NBUF, CHUNK * K, C), jnp.float32),
        pltpu.VMEM((2, OROWS, C), jnp.float32),
        pltpu.VMEM_SHARED((N, C), jnp.float32),
        pltpu.SemaphoreType.DMA,
        pltpu.SemaphoreType.DMA,
        pltpu.SemaphoreType.DMA,
        pltpu.SemaphoreType.DMA,
    ],
)
def _sc_gather_max(x_hbm, e_hbm, out_hbm, e_v, rows_v, o_v, xs,
                   g0, g1, os0, os1):
    wid = lax.axis_index("s") * 2 + lax.axis_index("c")
    sid = lax.axis_index("s")
    rpw = 624
    pltpu.sync_copy(x_hbm.at[pl.ds(sid * rpw, rpw)], xs.at[pl.ds(sid * rpw, rpw)])

    @pl.when(sid == 0)
    def _():
        pltpu.sync_copy(x_hbm.at[pl.ds(16 * rpw, N - 16 * rpw)],
                        xs.at[pl.ds(16 * rpw, N - 16 * rpw)])
    pltpu.sync_copy(e_hbm.at[pl.ds(wid * NCHUNK, NCHUNK)], e_v)
    plsc.subcore_barrier()
    gsems = (g0, g1)
    osems = (os0, os1)

    def gather(step, buf):
        return pltpu.make_async_copy(
            xs.at[e_v.at[step]], rows_v.at[buf], gsems[buf])

    def flush(octet, ob):
        base = wid * NPW + octet * OROWS
        return pltpu.make_async_copy(
            o_v.at[ob], out_hbm.at[pl.ds(base, OROWS)], osems[ob])

    def flush_valid(octet):
        return wid * NPW + octet * OROWS + OROWS <= N

    for b in range(NBUF):
        gather(b, b).start()

    def compute(step, buf, orow):
        for n in range(CHUNK):
            accs = tuple(
                jnp.maximum(rows_v[buf, n * K, pl.ds(cb * LANES, LANES)],
                            rows_v[buf, n * K + 1, pl.ds(cb * LANES, LANES)])
                for cb in range(CB))

            def body(g, accs, n=n, buf=buf):
                base = n * K + 2 + g * 5
                new = []
                for cb, a in enumerate(accs):
                    for kk in range(5):
                        a = jnp.maximum(
                            a, rows_v[buf, base + kk, pl.ds(cb * LANES, LANES)])
                    new.append(a)
                return tuple(new)

            accs = lax.fori_loop(0, (K - 2) // 5, body, accs)
            for cb in range(CB):
                o_v[orow // OROWS, orow % OROWS + n, pl.ds(cb * LANES, LANES)] = accs[cb]

    def loop_body(gg, carry):
        for h in range(2):
            octet = gg * 2 + h

            @pl.when(jnp.logical_and(gg >= 1, flush_valid(octet - 2)))
            def _(h=h, octet=octet):
                flush(octet - 2, h).wait()

            for j in range(OSTEPS):
                step = octet * OSTEPS + j
                buf = (4 * h + j) % NBUF
                gather(step, buf).wait()
                compute(step, buf, h * OROWS + j * CHUNK)
                nxt = step + NBUF

                @pl.when(nxt < NCHUNK)
                def _(nxt=nxt, buf=buf):
                    gather(nxt, buf).start()

            @pl.when(flush_valid(octet))
            def _(h=h, octet=octet):
                flush(octet, h).start()
        return carry

    lax.fori_loop(0, NCHUNK // (2 * OSTEPS), loop_body, 0)
    for tail in (NCHUNK // OSTEPS - 2, NCHUNK // OSTEPS - 1):
        @pl.when(flush_valid(tail))
        def _(tail=tail):
            flush(tail, tail % 2).wait()


def _erf(z):
    a1, a2, a3, a4, a5 = (0.254829592, -0.284496736, 1.421413741,
                          -1.453152027, 1.061405429)
    p = 0.3275911
    s = jnp.sign(z)
    az = jnp.abs(z)
    t = 1.0 / (1.0 + p * az)
    poly = ((((a5 * t + a4) * t + a3) * t + a2) * t + a1) * t
    return s * (1.0 - poly * jnp.exp(-az * az))


def _tc_mlp_body(x_ref, g_ref, wd_ref, w2_ref, b_ref, gm_ref, bt_ref, o_ref):
    h = (jnp.dot(x_ref[...], wd_ref[...], preferred_element_type=jnp.float32)
         + jnp.dot(g_ref[...], w2_ref[...], preferred_element_type=jnp.float32)
         + b_ref[...])
    mean = jnp.mean(h, axis=0, keepdims=True)
    var = jnp.mean((h - mean) ** 2, axis=0, keepdims=True)
    hn = (h - mean) * lax.rsqrt(var + 1e-5) * gm_ref[...] + bt_ref[...]
    o_ref[...] = 0.5 * hn * (1.0 + _erf(hn * (1.0 / math.sqrt(2.0))))


def kernel(x, edge_index, W, b, gamma, beta):
    xf = x[0]
    e = edge_index[0]
    e_pad = jnp.concatenate(
        [e, jnp.zeros((NPAD - N, K), jnp.int32)], axis=0)
    e2 = e_pad.reshape(NW * NCHUNK, CHUNK * K)

    maxg = _sc_gather_max(xf, e2)

    wd = W[:C] - W[C:]
    w2 = W[C:]
    out = pl.pallas_call(
        _tc_mlp_body,
        out_shape=jax.ShapeDtypeStruct((N, COUT), jnp.float32),
    )(xf, maxg, wd, w2, b.reshape(1, COUT), gamma.reshape(1, COUT),
      beta.reshape(1, COUT))
    return out.reshape(1, N, COUT)


# speedup vs baseline: 24.0633x; 1.0232x over previous
import functools
import math

import jax
import jax.numpy as jnp
from jax import lax
from jax.experimental import pallas as pl
from jax.experimental.pallas import tpu as pltpu
from jax.experimental.pallas import tpu_sc as plsc

N = 10000
K = 32
C = 128
COUT = 128

NW = 32
NPW = 320
NPAD = NW * NPW
CHUNK = 2
NCHUNK = NPW // CHUNK
NBUF = 2
OROWS = 8
OSTEPS = OROWS // CHUNK
LANES = 16
CB = C // LANES

_sc_mesh = plsc.VectorSubcoreMesh(core_axis_name="c", subcore_axis_name="s")


@functools.partial(
    pl.kernel,
    mesh=_sc_mesh,
    out_type=jax.ShapeDtypeStruct((N, C), jnp.float32),
    scratch_types=[
        pltpu.VMEM((NCHUNK, CHUNK * K), jnp.int32),
        pltpu.VMEM((NBUF, CHUNK * K, C), jnp.float32),
        pltpu.VMEM((2, OROWS, C), jnp.float32),
        pltpu.VMEM_SHARED((N, C), jnp.float32),
        pltpu.SemaphoreType.DMA,
        pltpu.SemaphoreType.DMA,
        pltpu.SemaphoreType.DMA,
        pltpu.SemaphoreType.DMA,
    ],
)
def _sc_gather_max(x_hbm, e_hbm, out_hbm, e_v, rows_v, o_v, xs,
                   g0, g1, os0, os1):
    wid = lax.axis_index("s") * 2 + lax.axis_index("c")
    sid = lax.axis_index("s")
    rpw = 624
    pltpu.sync_copy(x_hbm.at[pl.ds(sid * rpw, rpw)], xs.at[pl.ds(sid * rpw, rpw)])

    @pl.when(sid == 0)
    def _():
        pltpu.sync_copy(x_hbm.at[pl.ds(16 * rpw, N - 16 * rpw)],
                        xs.at[pl.ds(16 * rpw, N - 16 * rpw)])
    pltpu.sync_copy(e_hbm.at[pl.ds(wid * NCHUNK, NCHUNK)], e_v)
    plsc.subcore_barrier()
    gsems = (g0, g1)
    osems = (os0, os1)

    def gather(step, buf):
        return pltpu.make_async_copy(
            xs.at[e_v.at[step]], rows_v.at[buf], gsems[buf])

    def flush(octet, ob):
        base = wid * NPW + octet * OROWS
        return pltpu.make_async_copy(
            o_v.at[ob], out_hbm.at[pl.ds(base, OROWS)], osems[ob])

    def flush_valid(octet):
        return wid * NPW + octet * OROWS + OROWS <= N

    for b in range(NBUF):
        gather(b, b).start()

    def compute(step, buf, orow):
        for n in range(CHUNK):
            accs = tuple(
                jnp.maximum(rows_v[buf, n * K, pl.ds(cb * LANES, LANES)],
                            rows_v[buf, n * K + 1, pl.ds(cb * LANES, LANES)])
                for cb in range(CB))

            def body(g, accs, n=n, buf=buf):
                base = n * K + 2 + g * 5
                new = []
                for cb, a in enumerate(accs):
                    for kk in range(5):
                        a = jnp.maximum(
                            a, rows_v[buf, base + kk, pl.ds(cb * LANES, LANES)])
                    new.append(a)
                return tuple(new)

            accs = lax.fori_loop(0, (K - 2) // 5, body, accs)
            for cb in range(CB):
                o_v[orow // OROWS, orow % OROWS + n, pl.ds(cb * LANES, LANES)] = accs[cb]

    def loop_body(gg, carry):
        for h in range(2):
            octet = gg * 2 + h

            @pl.when(jnp.logical_and(gg >= 1, flush_valid(octet - 2)))
            def _(h=h, octet=octet):
                flush(octet - 2, h).wait()

            for j in range(OSTEPS):
                step = octet * OSTEPS + j
                buf = (4 * h + j) % NBUF
                gather(step, buf).wait()
                compute(step, buf, h * OROWS + j * CHUNK)
                nxt = step + NBUF

                @pl.when(nxt < NCHUNK)
                def _(nxt=nxt, buf=buf):
                    gather(nxt, buf).start()

            @pl.when(flush_valid(octet))
            def _(h=h, octet=octet):
                flush(octet, h).start()
        return carry

    lax.fori_loop(0, NCHUNK // (2 * OSTEPS), loop_body, 0)
    for tail in (NCHUNK // OSTEPS - 2, NCHUNK // OSTEPS - 1):
        @pl.when(flush_valid(tail))
        def _(tail=tail):
            flush(tail, tail % 2).wait()


def _erf(z):
    a1, a2, a3, a4, a5 = (0.254829592, -0.284496736, 1.421413741,
                          -1.453152027, 1.061405429)
    p = 0.3275911
    s = jnp.sign(z)
    az = jnp.abs(z)
    t = 1.0 / (1.0 + p * az)
    poly = ((((a5 * t + a4) * t + a3) * t + a2) * t + a1) * t
    return s * (1.0 - poly * jnp.exp(-az * az))


BLK = 1000
NBLK = N // BLK


def _tc_mlp_body(x_ref, g_ref, wd_ref, w2_ref, b_ref, gm_ref, bt_ref, o_ref,
                 h_ref, s_ref):
    p = pl.program_id(0)
    j = pl.program_id(1)

    @pl.when(p == 0)
    def _():
        h = (jnp.dot(x_ref[...], wd_ref[...], preferred_element_type=jnp.float32)
             + jnp.dot(g_ref[...], w2_ref[...], preferred_element_type=jnp.float32)
             + b_ref[...])
        h_ref[pl.ds(j * BLK, BLK), :] = h

        @pl.when(j == 0)
        def _():
            s_ref[...] = jnp.zeros_like(s_ref)

        s_ref[0:1, :] += jnp.sum(h, axis=0, keepdims=True)
        s_ref[1:2, :] += jnp.sum(h * h, axis=0, keepdims=True)

    @pl.when(p == 1)
    def _():
        h = h_ref[pl.ds(j * BLK, BLK), :]
        mean = s_ref[0:1, :] * (1.0 / N)
        var = s_ref[1:2, :] * (1.0 / N) - mean * mean
        hn = (h - mean) * lax.rsqrt(var + 1e-5) * gm_ref[...] + bt_ref[...]
        o_ref[...] = 0.5 * hn * (1.0 + _erf(hn * (1.0 / math.sqrt(2.0))))


def kernel(x, edge_index, W, b, gamma, beta):
    xf = x[0]
    e = edge_index[0]
    e_pad = jnp.concatenate(
        [e, jnp.zeros((NPAD - N, K), jnp.int32)], axis=0)
    e2 = e_pad.reshape(NW * NCHUNK, CHUNK * K)

    maxg = _sc_gather_max(xf, e2)

    wd = W[:C] - W[C:]
    w2 = W[C:]
    row_spec = pl.BlockSpec((BLK, C), lambda p, j: ((1 - p) * j, 0))
    full_spec = pl.BlockSpec((C, COUT), lambda p, j: (0, 0))
    vec_spec = pl.BlockSpec((1, COUT), lambda p, j: (0, 0))
    out = pl.pallas_call(
        _tc_mlp_body,
        grid=(2, NBLK),
        in_specs=[row_spec, row_spec, full_spec, full_spec,
                  vec_spec, vec_spec, vec_spec],
        out_specs=pl.BlockSpec((BLK, COUT), lambda p, j: (p * j, 0)),
        scratch_shapes=[pltpu.VMEM((N, COUT), jnp.float32),
                        pltpu.VMEM((8, COUT), jnp.float32)],
        out_shape=jax.ShapeDtypeStruct((N, COUT), jnp.float32),
    )(xf, maxg, wd, w2, b.reshape(1, COUT), gamma.reshape(1, COUT),
      beta.reshape(1, COUT))
    return out.reshape(1, N, COUT)


# speedup vs baseline: 24.3023x; 1.0099x over previous
import functools
import math

import jax
import jax.numpy as jnp
from jax import lax
from jax.experimental import pallas as pl
from jax.experimental.pallas import tpu as pltpu
from jax.experimental.pallas import tpu_sc as plsc

N = 10000
K = 32
C = 128
COUT = 128

NW = 32
NPW = 336
NPAD = NW * NPW
CHUNK = 2
NCHUNK = NPW // CHUNK
NBUF = 3
OROWS = 8
OSTEPS = OROWS // CHUNK
LANES = 16
CB = C // LANES

_sc_mesh = plsc.VectorSubcoreMesh(core_axis_name="c", subcore_axis_name="s")


@functools.partial(
    pl.kernel,
    mesh=_sc_mesh,
    out_type=jax.ShapeDtypeStruct((N, C), jnp.float32),
    scratch_types=[
        pltpu.VMEM((NCHUNK, CHUNK * K), jnp.int32),
        pltpu.VMEM((NBUF, CHUNK * K, C), jnp.float32),
        pltpu.VMEM((2, OROWS, C), jnp.float32),
        pltpu.VMEM_SHARED((N, C), jnp.float32),
        pltpu.SemaphoreType.DMA,
        pltpu.SemaphoreType.DMA,
        pltpu.SemaphoreType.DMA,
        pltpu.SemaphoreType.DMA,
        pltpu.SemaphoreType.DMA,
    ],
)
def _sc_gather_max(x_hbm, e_hbm, out_hbm, e_v, rows_v, o_v, xs,
                   g0, g1, g2, os0, os1):
    wid = lax.axis_index("s") * 2 + lax.axis_index("c")
    sid = lax.axis_index("s")
    rpw = 624
    pltpu.sync_copy(x_hbm.at[pl.ds(sid * rpw, rpw)], xs.at[pl.ds(sid * rpw, rpw)])

    @pl.when(sid == 0)
    def _():
        pltpu.sync_copy(x_hbm.at[pl.ds(16 * rpw, N - 16 * rpw)],
                        xs.at[pl.ds(16 * rpw, N - 16 * rpw)])
    pltpu.sync_copy(e_hbm.at[pl.ds(wid * NCHUNK, NCHUNK)], e_v)
    plsc.subcore_barrier()
    gsems = (g0, g1, g2)
    osems = (os0, os1)

    def gather(step, buf):
        return pltpu.make_async_copy(
            xs.at[e_v.at[step]], rows_v.at[buf], gsems[buf])

    def flush(octet, ob):
        base = wid * NPW + octet * OROWS
        return pltpu.make_async_copy(
            o_v.at[ob], out_hbm.at[pl.ds(base, OROWS)], osems[ob])

    def flush_valid(octet):
        return wid * NPW + octet * OROWS + OROWS <= N

    active = wid * NPW < N

    @pl.when(active)
    def _():
        for b in range(NBUF - 1):
            gather(b, b).start()

    def compute(step, buf, orow):
        for n in range(CHUNK):
            accs = tuple(
                jnp.maximum(rows_v[buf, n * K, pl.ds(cb * LANES, LANES)],
                            rows_v[buf, n * K + 1, pl.ds(cb * LANES, LANES)])
                for cb in range(CB))

            def body(g, accs, n=n, buf=buf):
                base = n * K + 2 + g * 5
                new = []
                for cb, a in enumerate(accs):
                    for kk in range(5):
                        a = jnp.maximum(
                            a, rows_v[buf, base + kk, pl.ds(cb * LANES, LANES)])
                    new.append(a)
                return tuple(new)

            accs = lax.fori_loop(0, (K - 2) // 5, body, accs)
            for cb in range(CB):
                o_v[orow // OROWS, orow % OROWS + n, pl.ds(cb * LANES, LANES)] = accs[cb]

    def loop_body(so, carry):
        for q in range(6):
            octet = so * 6 + q
            h = q % 2

            @pl.when(jnp.logical_and(octet >= 2, flush_valid(octet - 2)))
            def _(h=h, octet=octet):
                flush(octet - 2, h).wait()

            for j in range(OSTEPS):
                t = q * OSTEPS + j
                step = so * 24 + t
                gather(step, t % NBUF).wait()
                nxt = step + NBUF - 1

                @pl.when(nxt < NCHUNK)
                def _(nxt=nxt, b=(t + NBUF - 1) % NBUF):
                    gather(nxt, b).start()
                compute(step, t % NBUF, h * OROWS + j * CHUNK)

            @pl.when(flush_valid(octet))
            def _(h=h, octet=octet):
                flush(octet, h).start()
        return carry

    @pl.when(active)
    def _():
        lax.fori_loop(0, NCHUNK // 24, loop_body, 0)
        for tail in (NCHUNK // OSTEPS - 2, NCHUNK // OSTEPS - 1):
            @pl.when(flush_valid(tail))
            def _(tail=tail):
                flush(tail, tail % 2).wait()


def _erf(z):
    a1, a2, a3, a4, a5 = (0.254829592, -0.284496736, 1.421413741,
                          -1.453152027, 1.061405429)
    p = 0.3275911
    s = jnp.sign(z)
    az = jnp.abs(z)
    t = 1.0 / (1.0 + p * az)
    poly = ((((a5 * t + a4) * t + a3) * t + a2) * t + a1) * t
    return s * (1.0 - poly * jnp.exp(-az * az))


BLK = 1000
NBLK = N // BLK


def _tc_mlp_body(x_ref, g_ref, wd_ref, w2_ref, b_ref, gm_ref, bt_ref, o_ref,
                 h_ref, s_ref):
    p = pl.program_id(0)
    j = pl.program_id(1)

    @pl.when(p == 0)
    def _():
        h = (jnp.dot(x_ref[...], wd_ref[...], preferred_element_type=jnp.float32)
             + jnp.dot(g_ref[...], w2_ref[...], preferred_element_type=jnp.float32)
             + b_ref[...])
        h_ref[pl.ds(j * BLK, BLK), :] = h

        @pl.when(j == 0)
        def _():
            s_ref[...] = jnp.zeros_like(s_ref)

        s_ref[0:1, :] += jnp.sum(h, axis=0, keepdims=True)
        s_ref[1:2, :] += jnp.sum(h * h, axis=0, keepdims=True)

    @pl.when(p == 1)
    def _():
        h = h_ref[pl.ds(j * BLK, BLK), :]
        mean = s_ref[0:1, :] * (1.0 / N)
        var = s_ref[1:2, :] * (1.0 / N) - mean * mean
        hn = (h - mean) * lax.rsqrt(var + 1e-5) * gm_ref[...] + bt_ref[...]
        o_ref[...] = 0.5 * hn * (1.0 + _erf(hn * (1.0 / math.sqrt(2.0))))


def kernel(x, edge_index, W, b, gamma, beta):
    xf = x[0]
    e = edge_index[0]
    e_pad = jnp.concatenate(
        [e, jnp.zeros((NPAD - N, K), jnp.int32)], axis=0)
    e2 = e_pad.reshape(NW * NCHUNK, CHUNK * K)

    maxg = _sc_gather_max(xf, e2)

    wd = W[:C] - W[C:]
    w2 = W[C:]
    row_spec = pl.BlockSpec((BLK, C), lambda p, j: ((1 - p) * j, 0))
    full_spec = pl.BlockSpec((C, COUT), lambda p, j: (0, 0))
    vec_spec = pl.BlockSpec((1, COUT), lambda p, j: (0, 0))
    out = pl.pallas_call(
        _tc_mlp_body,
        grid=(2, NBLK),
        in_specs=[row_spec, row_spec, full_spec, full_spec,
                  vec_spec, vec_spec, vec_spec],
        out_specs=pl.BlockSpec((BLK, COUT), lambda p, j: (p * j, 0)),
        scratch_shapes=[pltpu.VMEM((N, COUT), jnp.float32),
                        pltpu.VMEM((8, COUT), jnp.float32)],
        out_shape=jax.ShapeDtypeStruct((N, COUT), jnp.float32),
    )(xf, maxg, wd, w2, b.reshape(1, COUT), gamma.reshape(1, COUT),
      beta.reshape(1, COUT))
    return out.reshape(1, N, COUT)
